# baseline (speedup 1.0000x reference)
import functools

import jax
import jax.numpy as jnp
from jax import lax
from jax.experimental import pallas as pl
from jax.experimental.pallas import tpu as pltpu
from jax.experimental.pallas import tpu_sc as plsc

_NC = 2
_NS = 16
_NW = _NC * _NS


def _sc_copy(total, per_w, chunk, nch, in_hbm, out_hbm,
             buf0, buf1, rs0, rs1, ws0, ws1):
    wid = lax.axis_index("s") * _NC + lax.axis_index("c")
    base = wid * per_w
    bufs = (buf0, buf1)
    rsem = (rs0, rs1)
    wsem = (ws0, ws1)

    def rd(j):
        b = j % 2
        return pltpu.make_async_copy(
            in_hbm.at[pl.ds(base + j * chunk, chunk)], bufs[b], rsem[b])

    def wr(j):
        b = j % 2
        return pltpu.make_async_copy(
            bufs[b], out_hbm.at[pl.ds(base + j * chunk, chunk)], wsem[b])

    rd(0).start()
    for j in range(nch):
        rd(j).wait()
        if j + 1 < nch:
            if j >= 1:
                wr(j - 1).wait()
            rd(j + 1).start()
        wr(j).start()
    wr(nch - 2).wait()
    wr(nch - 1).wait()


def kernel(table):
    n, d = table.shape
    flat = table.reshape(n * d)
    total = flat.shape[0]
    per_w = total // _NW
    chunk = 50_000
    nch = per_w // chunk

    mesh = plsc.VectorSubcoreMesh(core_axis_name="c", subcore_axis_name="s")
    run = pl.kernel(
        functools.partial(_sc_copy, total, per_w, chunk, nch),
        out_type=jax.ShapeDtypeStruct((total,), flat.dtype),
        mesh=mesh,
        scratch_types=[
            pltpu.VMEM((chunk,), jnp.float32),
            pltpu.VMEM((chunk,), jnp.float32),
            pltpu.SemaphoreType.DMA,
            pltpu.SemaphoreType.DMA,
            pltpu.SemaphoreType.DMA,
            pltpu.SemaphoreType.DMA,
        ],
    )
    return run(flat).reshape(n, d)


# speedup vs baseline: 1.2265x; 1.2265x over previous
import jax
import jax.numpy as jnp
from jax.experimental import pallas as pl


def _copy_block(in_ref, out_ref):
    out_ref[...] = in_ref[...]


def kernel(table):
    n, d = table.shape
    block = 20000
    return pl.pallas_call(
        _copy_block,
        grid=(n // block,),
        in_specs=[pl.BlockSpec((block, d), lambda i: (i, 0))],
        out_specs=pl.BlockSpec((block, d), lambda i: (i, 0)),
        out_shape=jax.ShapeDtypeStruct((n, d), table.dtype),
    )(table)
